# Initial kernel scaffold; baseline (speedup 1.0000x reference)
import jax
import jax.numpy as jnp
from jax.experimental import pallas as pl


def kernel(x, weight, lora_A, lora_B_q, lora_B_k, lora_B_v):
    raise NotImplementedError("write your pallas kernel here")


# speedup vs baseline: 1.5267x; 1.5267x over previous
import jax
import jax.numpy as jnp
from jax.experimental import pallas as pl

_HIDDEN = 2048
_OUT = 3072
_Q = 2048
_KV = 512
_R = 16
_SCALING = 2.0
_TM = 512


def _fused_kernel(x_ref, w_ref, a_ref, b_ref, o_ref):
    xt = x_ref[...]
    dn = (((1,), (1,)), ((), ()))
    base = jax.lax.dot_general(xt, w_ref[...], dn,
                               preferred_element_type=jnp.float32)
    xa = jax.lax.dot_general(xt, a_ref[...], dn,
                             preferred_element_type=jnp.float32)
    delta = jax.lax.dot_general(xa, b_ref[...], dn,
                                preferred_element_type=jnp.float32)
    o_ref[...] = base + delta * _SCALING


def kernel(x, weight, lora_A, lora_B_q, lora_B_k, lora_B_v):
    orig_shape = x.shape
    x_flat = x.reshape(-1, _HIDDEN)
    n = x_flat.shape[0]

    a48 = lora_A[0].reshape(3 * _R, _HIDDEN)
    bbd = jnp.zeros((_OUT, 3 * _R), dtype=jnp.float32)
    bbd = bbd.at[:_Q, :_R].set(lora_B_q[0])
    bbd = bbd.at[_Q:_Q + _KV, _R:2 * _R].set(lora_B_k[0])
    bbd = bbd.at[_Q + _KV:, 2 * _R:].set(lora_B_v[0])

    grid = (n // _TM,)
    out = pl.pallas_call(
        _fused_kernel,
        grid=grid,
        in_specs=[
            pl.BlockSpec((_TM, _HIDDEN), lambda i: (i, 0)),
            pl.BlockSpec((_OUT, _HIDDEN), lambda i: (0, 0)),
            pl.BlockSpec((3 * _R, _HIDDEN), lambda i: (0, 0)),
            pl.BlockSpec((_OUT, 3 * _R), lambda i: (0, 0)),
        ],
        out_specs=pl.BlockSpec((_TM, _OUT), lambda i: (i, 0)),
        out_shape=jax.ShapeDtypeStruct((n, _OUT), jnp.float32),
    )(x_flat, weight, a48, bbd)
    return out.reshape(*orig_shape[:-1], _OUT)
